# baseline (speedup 1.0000x reference)
import functools

import jax
import jax.numpy as jnp
from jax import lax
from jax.experimental import pallas as pl
from jax.experimental.pallas import tpu as pltpu
from jax.experimental.pallas import tpu_sc as plsc

N = 8192
H = 2048
E = 8
F = 4096

NC, NS, L = 2, 16, 16
NW = NC * NS
CHUNK = N // NW

BM = 256
P = TOTAL = N * 2 + E * BM
NB = P // BM
PX = P + 16
DUMP = P
NBPAD = 80

RPW = P // NW
CG = 16

_MESH = plsc.VectorSubcoreMesh(core_axis_name="c", subcore_axis_name="s")

_SC_PARAMS = pltpu.CompilerParams(
    needs_layout_passes=False, use_tc_tiling_on_sc=False)


def _wid():
    return lax.axis_index("s") * NC + lax.axis_index("c")


def _router_body(x_ref, w_ref, b_ref, lt_ref, xbf_ref):
    xb = x_ref[...]
    lt = lax.dot_general(w_ref[...], xb, (((0,), (1,)), ((), ())),
                         preferred_element_type=jnp.float32)
    lt_ref[...] = lt + b_ref[...]
    xbf_ref[...] = xb.astype(jnp.bfloat16)


def _router(flat, router_w, router_b):
    bmr = 1024
    return pl.pallas_call(
        _router_body,
        grid=(N // bmr,),
        in_specs=[
            pl.BlockSpec((bmr, H), lambda i: (i, 0)),
            pl.BlockSpec((H, E), lambda i: (0, 0)),
            pl.BlockSpec((E, 1), lambda i: (0, 0)),
        ],
        out_specs=[
            pl.BlockSpec((E, bmr), lambda i: (0, i)),
            pl.BlockSpec((bmr, H), lambda i: (i, 0)),
        ],
        out_shape=[
            jax.ShapeDtypeStruct((E, N), jnp.float32),
            jax.ShapeDtypeStruct((N, H), jnp.bfloat16),
        ],
    )(flat, router_w, router_b.reshape(E, 1))


def _route_body(lt_hbm, eidx_hbm, probs_hbm, hist_hbm,
                lt_v, e0_v, e1_v, p0_v, p1_v, h_v):
    wid = _wid()
    base = pl.multiple_of(wid * CHUNK, CHUNK)
    pltpu.sync_copy(lt_hbm.at[:, pl.ds(base, CHUNK)], lt_v)
    lanes = lax.iota(jnp.int32, L)
    cnt = [jnp.zeros((L,), jnp.int32) for _ in range(E)]
    for g in range(CHUNK // L):
        sl = pl.ds(g * L, L)
        m1 = lt_v[0, sl]
        i1 = jnp.zeros((L,), jnp.int32)
        m2 = jnp.full((L,), -jnp.inf, jnp.float32)
        i2 = jnp.zeros((L,), jnp.int32)
        for e in range(1, E):
            v = lt_v[e, sl]
            ev = jnp.full((L,), e, jnp.int32)
            gt = v > m1
            g2 = v > m2
            m2n = jnp.where(gt, m1, jnp.where(g2, v, m2))
            i2n = jnp.where(gt, i1, jnp.where(g2, ev, i2))
            m1 = jnp.where(gt, v, m1)
            i1 = jnp.where(gt, ev, i1)
            m2, i2 = m2n, i2n
        d = jnp.exp(m2 - m1)
        p0 = 1.0 / (1.0 + d)
        e0_v[sl] = i1
        e1_v[sl] = i2
        p0_v[sl] = p0
        p1_v[sl] = 1.0 - p0
        for e in range(E):
            cnt[e] = (cnt[e] + (i1 == e).astype(jnp.int32)
                      + (i2 == e).astype(jnp.int32))
    hv = jnp.zeros((L,), jnp.int32)
    for e in range(E):
        hv = jnp.where(lanes == e, jnp.sum(cnt[e]), hv)
    h_v[...] = hv
    pltpu.sync_copy(e0_v, eidx_hbm.at[0, pl.ds(base, CHUNK)])
    pltpu.sync_copy(e1_v, eidx_hbm.at[1, pl.ds(base, CHUNK)])
    pltpu.sync_copy(p0_v, probs_hbm.at[0, pl.ds(base, CHUNK)])
    pltpu.sync_copy(p1_v, probs_hbm.at[1, pl.ds(base, CHUNK)])
    pltpu.sync_copy(h_v.at[pl.ds(0, E)], hist_hbm.at[wid])


_route = functools.partial(
    pl.kernel,
    _route_body,
    out_type=[
        jax.ShapeDtypeStruct((2, N), jnp.int32),
        jax.ShapeDtypeStruct((2, N), jnp.float32),
        jax.ShapeDtypeStruct((NW, E), jnp.int32),
    ],
    mesh=_MESH,
    compiler_params=_SC_PARAMS,
    scratch_types=[
        pltpu.VMEM((E, CHUNK), jnp.float32),
        pltpu.VMEM((CHUNK,), jnp.int32),
        pltpu.VMEM((CHUNK,), jnp.int32),
        pltpu.VMEM((CHUNK,), jnp.float32),
        pltpu.VMEM((CHUNK,), jnp.float32),
        pltpu.VMEM((L,), jnp.int32),
    ],
)()


def _dispatch_body(eidx_hbm, probs_hbm, hist_hbm,
                   rt_hbm, rw8_hbm, cs_hbm, be_hbm,
                   hist_v, e0_v, e1_v, p0_v, p1_v,
                   base_v, gof_v, cnt_v, s0_v, s1_v,
                   tok_v, si_v, w8_v, padi_v, padt_v, padw_v, be_v, sem, semw):
    wid = _wid()
    base = pl.multiple_of(wid * CHUNK, CHUNK)
    lanes = lax.iota(jnp.int32, L)
    zeros_i = jnp.zeros((L,), jnp.int32)
    zeros_f = jnp.zeros((L,), jnp.float32)

    pltpu.sync_copy(hist_hbm, hist_v)
    pltpu.sync_copy(eidx_hbm.at[0, pl.ds(base, CHUNK)], e0_v)
    pltpu.sync_copy(eidx_hbm.at[1, pl.ds(base, CHUNK)], e1_v)
    pltpu.sync_copy(probs_hbm.at[0, pl.ds(base, CHUNK)], p0_v)
    pltpu.sync_copy(probs_hbm.at[1, pl.ds(base, CHUNK)], p1_v)

    go = jnp.int32(0)
    base_vec = zeros_i
    go_vec = zeros_i
    cnt_vec = zeros_i
    for e in range(E):
        esp = jnp.full((L,), e, jnp.int32)
        c0 = plsc.load_gather(hist_v, [lanes, esp])
        c1 = plsc.load_gather(hist_v, [lanes + L, esp])
        tot = jnp.sum(c0) + jnp.sum(c1)
        prior = (jnp.sum(jnp.where(lanes < wid, c0, 0))
                 + jnp.sum(jnp.where(lanes + L < wid, c1, 0)))
        go_vec = jnp.where(lanes == e, go, go_vec)
        cnt_vec = jnp.where(lanes == e, tot, cnt_vec)
        base_vec = jnp.where(lanes == e, go + prior, base_vec)
        go = go + ((tot + (BM - 1)) // BM) * BM
    go_vec = jnp.where(lanes == E, go, go_vec)
    base_v[...] = base_vec
    gof_v[...] = go_vec
    cnt_v[...] = cnt_vec

    for g in range(CHUNK // L):
        sl = pl.ds(g * L, L)
        for k in range(2):
            ev = e0_v[sl] if k == 0 else e1_v[sl]
            pv = p0_v[sl] if k == 0 else p1_v[sl]
            bv = plsc.load_gather(base_v, [ev])
            rank = zeros_i
            upd = zeros_i
            for e in range(E):
                mi = (ev == e).astype(jnp.int32)
                incl = plsc.cumsum(mi)
                rank = jnp.where(ev == e, incl - 1, rank)
                upd = jnp.where(lanes == e, jnp.sum(mi), upd)
            pos = bv + rank
            base_vec = base_vec + upd
            base_v[...] = base_vec
            if k == 0:
                s0_v[sl] = pos
            else:
                s1_v[sl] = pos
            o = g * 2 * L + k * L
            si_v[o // 64, pl.ds(o % 64, L)] = pos
            tok_v[o // 64, pl.ds(o % 64, L)] = base + g * L + lanes
            plsc.store_scatter(w8_v, [o + lanes, zeros_i], pv)

    pltpu.sync_copy(s0_v, cs_hbm.at[0, pl.ds(base, CHUNK)])
    pltpu.sync_copy(s1_v, cs_hbm.at[1, pl.ds(base, CHUNK)])
    hs = []
    for r in range(8):
        hs.append(pltpu.async_copy(tok_v.at[r], rt_hbm.at[si_v.at[r]], sem))
        hs.append(pltpu.async_copy(w8_v.at[pl.ds(r * 64, 64)],
                                   rw8_hbm.at[si_v.at[r]], semw))
    for h in hs:
        h.wait()

    dump = jnp.full((L,), DUMP, jnp.int32)
    widv = jnp.full((L,), 0, jnp.int32) + wid
    my_go = plsc.load_gather(gof_v, [jnp.minimum(widv, E)])
    my_go1 = plsc.load_gather(gof_v, [jnp.minimum(widv + 1, E)])
    my_cnt = plsc.load_gather(cnt_v, [jnp.minimum(widv, E - 1)])
    total_go = plsc.load_gather(gof_v, [jnp.full((L,), E, jnp.int32)])
    for q in range(16):
        idxv = dump
        tail_start = my_go + my_cnt + q * L + lanes
        tail_ok = (widv < E) & (tail_start < my_go1)
        idxv = jnp.where(tail_ok, tail_start, idxv)
        tr_start = total_go + (widv - E) * CHUNK + q * L + lanes
        tr_ok = (widv >= E) & (widv < 2 * E) & (tr_start < P)
        idxv = jnp.where(tr_ok, tr_start, idxv)
        padi_v[q // 4, pl.ds((q % 4) * L, L)] = idxv
        padt_v[q // 4, pl.ds((q % 4) * L, L)] = zeros_i
        plsc.store_scatter(padw_v, [q * L + lanes, zeros_i], zeros_f)
    hs2 = []
    for r in range(4):
        hs2.append(pltpu.async_copy(padt_v.at[r], rt_hbm.at[padi_v.at[r]], sem))
        hs2.append(pltpu.async_copy(padw_v.at[pl.ds(r * 64, 64)],
                                    rw8_hbm.at[padi_v.at[r]], semw))
    for h in hs2:
        h.wait()

    @pl.when(wid == L)
    def _():
        for q in range(NBPAD // L):
            bs = (q * L + lanes) * BM
            acc = jnp.zeros((L,), jnp.int32)
            for e in range(1, E + 1):
                goe = plsc.load_gather(gof_v, [jnp.full((L,), e, jnp.int32)])
                acc = acc + (bs >= goe).astype(jnp.int32)
            be_v[pl.ds(q * L, L)] = jnp.minimum(acc, E - 1)
        pltpu.sync_copy(be_v, be_hbm)


_dispatch = functools.partial(
    pl.kernel,
    _dispatch_body,
    out_type=[
        jax.ShapeDtypeStruct((PX,), jnp.int32),
        jax.ShapeDtypeStruct((PX, 8), jnp.float32),
        jax.ShapeDtypeStruct((2, N), jnp.int32),
        jax.ShapeDtypeStruct((NBPAD,), jnp.int32),
    ],
    mesh=_MESH,
    compiler_params=_SC_PARAMS,
    scratch_types=[
        pltpu.VMEM((NW, E), jnp.int32),
        pltpu.VMEM((CHUNK,), jnp.int32),
        pltpu.VMEM((CHUNK,), jnp.int32),
        pltpu.VMEM((CHUNK,), jnp.float32),
        pltpu.VMEM((CHUNK,), jnp.float32),
        pltpu.VMEM((L,), jnp.int32),
        pltpu.VMEM((L,), jnp.int32),
        pltpu.VMEM((L,), jnp.int32),
        pltpu.VMEM((CHUNK,), jnp.int32),
        pltpu.VMEM((CHUNK,), jnp.int32),
        pltpu.VMEM((8, 64), jnp.int32),
        pltpu.VMEM((8, 64), jnp.int32),
        pltpu.VMEM((512, 8), jnp.float32),
        pltpu.VMEM((4, 64), jnp.int32),
        pltpu.VMEM((4, 64), jnp.int32),
        pltpu.VMEM((256, 8), jnp.float32),
        pltpu.VMEM((NBPAD,), jnp.int32),
        pltpu.SemaphoreType.DMA,
        pltpu.SemaphoreType.DMA,
    ],
)()


GB = 32


def _stream_rows(src_hbm, idx_v, i_base, dst_hbm, d_base, n_rows,
                 buf0, buf1, s0, s1):
    def body(g, carry):
        o = g * (2 * GB)
        h0 = pltpu.async_copy(
            src_hbm.at[idx_v.at[pl.ds(i_base + o, GB)]], buf0, s0)
        h1 = pltpu.async_copy(
            src_hbm.at[idx_v.at[pl.ds(i_base + o + GB, GB)]], buf1, s1)
        h0.wait()
        pltpu.sync_copy(buf0, dst_hbm.at[pl.ds(d_base + o, GB)])
        h1.wait()
        pltpu.sync_copy(buf1, dst_hbm.at[pl.ds(d_base + o + GB, GB)])
        return carry

    lax.fori_loop(0, n_rows // (2 * GB), body, 0)


def _gather_body(xbf_hbm, rt_hbm, xg_hbm, idx_v, buf0, buf1, s0, s1):
    wid = _wid()
    r0 = pl.multiple_of(wid * RPW, CG)
    pltpu.sync_copy(rt_hbm.at[pl.ds(r0, RPW)], idx_v)
    _stream_rows(xbf_hbm, idx_v, 0, xg_hbm, r0, RPW, buf0, buf1, s0, s1)


_gather = functools.partial(
    pl.kernel,
    _gather_body,
    out_type=[jax.ShapeDtypeStruct((P, H), jnp.bfloat16)],
    mesh=_MESH,
    compiler_params=_SC_PARAMS,
    scratch_types=[
        pltpu.VMEM((RPW,), jnp.int32),
        pltpu.VMEM((GB, H), jnp.bfloat16),
        pltpu.VMEM((GB, H), jnp.bfloat16),
        pltpu.SemaphoreType.DMA,
        pltpu.SemaphoreType.DMA,
    ],
)()


def _gmm1_body(be_ref, x_ref, w_hbm, b_ref, o_ref, w_vmem, sem):
    i = pl.program_id(0)
    changed = (i == 0) | (be_ref[i] != be_ref[jnp.maximum(i - 1, 0)])

    @pl.when(changed)
    def _():
        cp = pltpu.make_async_copy(w_hbm.at[be_ref[i]], w_vmem, sem)
        cp.start()
        cp.wait()

    acc = lax.dot_general(x_ref[...], w_vmem[...], (((1,), (0,)), ((), ())),
                          preferred_element_type=jnp.float32)
    acc = acc + b_ref[0]
    o_ref[...] = jnp.maximum(acc, 0.0).astype(jnp.bfloat16)


def _gmm1(be, xg, w1b, b1):
    gs = pltpu.PrefetchScalarGridSpec(
        num_scalar_prefetch=1,
        grid=(NB,),
        in_specs=(
            pl.BlockSpec((BM, H), lambda i, be: (i, 0)),
            pl.BlockSpec(memory_space=pltpu.MemorySpace.HBM),
            pl.BlockSpec((1, 1, F), lambda i, be: (be[i], 0, 0)),
        ),
        out_specs=pl.BlockSpec((BM, F), lambda i, be: (i, 0)),
        scratch_shapes=[
            pltpu.VMEM((H, F), jnp.bfloat16),
            pltpu.SemaphoreType.DMA,
        ],
    )
    return pl.pallas_call(
        _gmm1_body, grid_spec=gs,
        out_shape=jax.ShapeDtypeStruct((P, F), jnp.bfloat16),
    )(be, xg, w1b, b1.reshape(E, 1, F))


def _gmm2_body(be_ref, x_ref, w_hbm, b_ref, rw_ref, o_ref, w_vmem, sem):
    i = pl.program_id(0)
    changed = (i == 0) | (be_ref[i] != be_ref[jnp.maximum(i - 1, 0)])

    @pl.when(changed)
    def _():
        cp = pltpu.make_async_copy(w_hbm.at[be_ref[i]], w_vmem, sem)
        cp.start()
        cp.wait()

    acc = lax.dot_general(x_ref[...], w_vmem[...], (((1,), (0,)), ((), ())),
                          preferred_element_type=jnp.float32)
    acc = acc + b_ref[0]
    acc = acc * rw_ref[:, 0:1]
    o_ref[...] = acc.astype(jnp.bfloat16)


def _gmm2(be, h1, w2b, b2, rw8):
    gs = pltpu.PrefetchScalarGridSpec(
        num_scalar_prefetch=1,
        grid=(NB,),
        in_specs=(
            pl.BlockSpec((BM, F), lambda i, be: (i, 0)),
            pl.BlockSpec(memory_space=pltpu.MemorySpace.HBM),
            pl.BlockSpec((1, 1, H), lambda i, be: (be[i], 0, 0)),
            pl.BlockSpec((BM, 8), lambda i, be: (i, 0)),
        ),
        out_specs=pl.BlockSpec((BM, H), lambda i, be: (i, 0)),
        scratch_shapes=[
            pltpu.VMEM((F, H), jnp.bfloat16),
            pltpu.SemaphoreType.DMA,
        ],
    )
    return pl.pallas_call(
        _gmm2_body, grid_spec=gs,
        out_shape=jax.ShapeDtypeStruct((P, H), jnp.bfloat16),
    )(be, h1, w2b, b2.reshape(E, 1, H), rw8)


def _combine_body(buf_hbm, cs_hbm, a_hbm, b_hbm, i0_v, i1_v,
                  buf0, buf1, s0, s1):
    wid = _wid()
    base = pl.multiple_of(wid * CHUNK, CHUNK)
    pltpu.sync_copy(cs_hbm.at[0, pl.ds(base, CHUNK)], i0_v)
    pltpu.sync_copy(cs_hbm.at[1, pl.ds(base, CHUNK)], i1_v)
    _stream_rows(buf_hbm, i0_v, 0, a_hbm, base, CHUNK, buf0, buf1, s0, s1)
    _stream_rows(buf_hbm, i1_v, 0, b_hbm, base, CHUNK, buf0, buf1, s0, s1)


_combine = functools.partial(
    pl.kernel,
    _combine_body,
    out_type=[
        jax.ShapeDtypeStruct((N, H), jnp.bfloat16),
        jax.ShapeDtypeStruct((N, H), jnp.bfloat16),
    ],
    mesh=_MESH,
    compiler_params=_SC_PARAMS,
    scratch_types=[
        pltpu.VMEM((CHUNK,), jnp.int32),
        pltpu.VMEM((CHUNK,), jnp.int32),
        pltpu.VMEM((GB, H), jnp.bfloat16),
        pltpu.VMEM((GB, H), jnp.bfloat16),
        pltpu.SemaphoreType.DMA,
        pltpu.SemaphoreType.DMA,
    ],
)()


def _add_body(a_ref, b_ref, o_ref):
    o_ref[...] = a_ref[...].astype(jnp.float32) + b_ref[...].astype(jnp.float32)


def _final_add(a, b):
    bmr = 512
    return pl.pallas_call(
        _add_body,
        grid=(N // bmr,),
        in_specs=[
            pl.BlockSpec((bmr, H), lambda i: (i, 0)),
            pl.BlockSpec((bmr, H), lambda i: (i, 0)),
        ],
        out_specs=pl.BlockSpec((bmr, H), lambda i: (i, 0)),
        out_shape=jax.ShapeDtypeStruct((N, H), jnp.float32),
    )(a, b)


def kernel(x, router_w, router_b, w1, b1, w2, b2):
    batch, seq, hidden = x.shape
    flat = x.reshape(N, H)
    lt, xbf = _router(flat, router_w, router_b)
    eidx, probs, hist = _route(lt)
    rt, rw8, cs, be = _dispatch(eidx, probs, hist)
    xg, = _gather(xbf, rt[:P])
    h1 = _gmm1(be, xg, w1.astype(jnp.bfloat16), b1)
    buf = _gmm2(be, h1, w2.astype(jnp.bfloat16), b2, rw8)
    a, b = _combine(buf, cs)
    out = _final_add(a, b)
    return out.reshape(batch, seq, hidden)


# speedup vs baseline: 1.0371x; 1.0371x over previous
import functools

import jax
import jax.numpy as jnp
from jax import lax
from jax.experimental import pallas as pl
from jax.experimental.pallas import tpu as pltpu
from jax.experimental.pallas import tpu_sc as plsc

N = 8192
H = 2048
E = 8
F = 4096

NC, NS, L = 2, 16, 16
NW = NC * NS
CHUNK = N // NW

BM = 256
P = TOTAL = N * 2 + E * BM
NB = P // BM
PX = P + 16
DUMP = P
NBPAD = 80

RPW = P // NW
CG = 16

_MESH = plsc.VectorSubcoreMesh(core_axis_name="c", subcore_axis_name="s")

_SC_PARAMS = pltpu.CompilerParams(
    needs_layout_passes=False, use_tc_tiling_on_sc=False)


def _wid():
    return lax.axis_index("s") * NC + lax.axis_index("c")


def _router_body(x_ref, w_ref, b_ref, lt_ref, xbf_ref):
    xb = x_ref[...]
    lt = lax.dot_general(w_ref[...], xb, (((0,), (1,)), ((), ())),
                         preferred_element_type=jnp.float32)
    lt_ref[...] = lt + b_ref[...]
    xbf_ref[...] = xb.astype(jnp.bfloat16)


def _router(flat, router_w, router_b):
    bmr = 1024
    return pl.pallas_call(
        _router_body,
        grid=(N // bmr,),
        in_specs=[
            pl.BlockSpec((bmr, H), lambda i: (i, 0)),
            pl.BlockSpec((H, E), lambda i: (0, 0)),
            pl.BlockSpec((E, 1), lambda i: (0, 0)),
        ],
        out_specs=[
            pl.BlockSpec((E, bmr), lambda i: (0, i)),
            pl.BlockSpec((bmr, H), lambda i: (i, 0)),
        ],
        out_shape=[
            jax.ShapeDtypeStruct((E, N), jnp.float32),
            jax.ShapeDtypeStruct((N, H), jnp.bfloat16),
        ],
    )(flat, router_w, router_b.reshape(E, 1))


def _route_body(lt_hbm, eidx_hbm, probs_hbm, hist_hbm,
                lt_v, e0_v, e1_v, p0_v, p1_v, h_v):
    wid = _wid()
    base = pl.multiple_of(wid * CHUNK, CHUNK)
    pltpu.sync_copy(lt_hbm.at[:, pl.ds(base, CHUNK)], lt_v)
    lanes = lax.iota(jnp.int32, L)
    cnt = [jnp.zeros((L,), jnp.int32) for _ in range(E)]
    for g in range(CHUNK // L):
        sl = pl.ds(g * L, L)
        m1 = lt_v[0, sl]
        i1 = jnp.zeros((L,), jnp.int32)
        m2 = jnp.full((L,), -jnp.inf, jnp.float32)
        i2 = jnp.zeros((L,), jnp.int32)
        for e in range(1, E):
            v = lt_v[e, sl]
            ev = jnp.full((L,), e, jnp.int32)
            gt = v > m1
            g2 = v > m2
            m2n = jnp.where(gt, m1, jnp.where(g2, v, m2))
            i2n = jnp.where(gt, i1, jnp.where(g2, ev, i2))
            m1 = jnp.where(gt, v, m1)
            i1 = jnp.where(gt, ev, i1)
            m2, i2 = m2n, i2n
        d = jnp.exp(m2 - m1)
        p0 = 1.0 / (1.0 + d)
        e0_v[sl] = i1
        e1_v[sl] = i2
        p0_v[sl] = p0
        p1_v[sl] = 1.0 - p0
        for e in range(E):
            cnt[e] = (cnt[e] + (i1 == e).astype(jnp.int32)
                      + (i2 == e).astype(jnp.int32))
    hv = jnp.zeros((L,), jnp.int32)
    for e in range(E):
        hv = jnp.where(lanes == e, jnp.sum(cnt[e]), hv)
    h_v[...] = hv
    pltpu.sync_copy(e0_v, eidx_hbm.at[0, pl.ds(base, CHUNK)])
    pltpu.sync_copy(e1_v, eidx_hbm.at[1, pl.ds(base, CHUNK)])
    pltpu.sync_copy(p0_v, probs_hbm.at[0, pl.ds(base, CHUNK)])
    pltpu.sync_copy(p1_v, probs_hbm.at[1, pl.ds(base, CHUNK)])
    pltpu.sync_copy(h_v.at[pl.ds(0, E)], hist_hbm.at[wid])


_route = functools.partial(
    pl.kernel,
    _route_body,
    out_type=[
        jax.ShapeDtypeStruct((2, N), jnp.int32),
        jax.ShapeDtypeStruct((2, N), jnp.float32),
        jax.ShapeDtypeStruct((NW, E), jnp.int32),
    ],
    mesh=_MESH,
    compiler_params=_SC_PARAMS,
    scratch_types=[
        pltpu.VMEM((E, CHUNK), jnp.float32),
        pltpu.VMEM((CHUNK,), jnp.int32),
        pltpu.VMEM((CHUNK,), jnp.int32),
        pltpu.VMEM((CHUNK,), jnp.float32),
        pltpu.VMEM((CHUNK,), jnp.float32),
        pltpu.VMEM((L,), jnp.int32),
    ],
)()


def _dispatch_body(eidx_hbm, probs_hbm, hist_hbm,
                   rt_hbm, rw8_hbm, cs_hbm, be_hbm,
                   hist_v, e0_v, e1_v, p0_v, p1_v,
                   base_v, gof_v, cnt_v, s0_v, s1_v,
                   tok_v, si_v, w8_v, padi_v, padt_v, padw_v, be_v, sem, semw):
    wid = _wid()
    base = pl.multiple_of(wid * CHUNK, CHUNK)
    lanes = lax.iota(jnp.int32, L)
    zeros_i = jnp.zeros((L,), jnp.int32)
    zeros_f = jnp.zeros((L,), jnp.float32)

    pltpu.sync_copy(hist_hbm, hist_v)
    pltpu.sync_copy(eidx_hbm.at[0, pl.ds(base, CHUNK)], e0_v)
    pltpu.sync_copy(eidx_hbm.at[1, pl.ds(base, CHUNK)], e1_v)
    pltpu.sync_copy(probs_hbm.at[0, pl.ds(base, CHUNK)], p0_v)
    pltpu.sync_copy(probs_hbm.at[1, pl.ds(base, CHUNK)], p1_v)

    go = jnp.int32(0)
    base_vec = zeros_i
    go_vec = zeros_i
    cnt_vec = zeros_i
    for e in range(E):
        esp = jnp.full((L,), e, jnp.int32)
        c0 = plsc.load_gather(hist_v, [lanes, esp])
        c1 = plsc.load_gather(hist_v, [lanes + L, esp])
        tot = jnp.sum(c0) + jnp.sum(c1)
        prior = (jnp.sum(jnp.where(lanes < wid, c0, 0))
                 + jnp.sum(jnp.where(lanes + L < wid, c1, 0)))
        go_vec = jnp.where(lanes == e, go, go_vec)
        cnt_vec = jnp.where(lanes == e, tot, cnt_vec)
        base_vec = jnp.where(lanes == e, go + prior, base_vec)
        go = go + ((tot + (BM - 1)) // BM) * BM
    go_vec = jnp.where(lanes == E, go, go_vec)
    base_v[...] = base_vec
    gof_v[...] = go_vec
    cnt_v[...] = cnt_vec

    for g in range(CHUNK // L):
        sl = pl.ds(g * L, L)
        for k in range(2):
            ev = e0_v[sl] if k == 0 else e1_v[sl]
            pv = p0_v[sl] if k == 0 else p1_v[sl]
            bv = plsc.load_gather(base_v, [ev])
            rank = zeros_i
            upd = zeros_i
            for e in range(E):
                mi = (ev == e).astype(jnp.int32)
                incl = plsc.cumsum(mi)
                rank = jnp.where(ev == e, incl - 1, rank)
                upd = jnp.where(lanes == e, jnp.sum(mi), upd)
            pos = bv + rank
            base_vec = base_vec + upd
            base_v[...] = base_vec
            if k == 0:
                s0_v[sl] = pos
            else:
                s1_v[sl] = pos
            o = g * 2 * L + k * L
            si_v[o // 64, pl.ds(o % 64, L)] = pos
            tok_v[o // 64, pl.ds(o % 64, L)] = base + g * L + lanes
            plsc.store_scatter(w8_v, [o + lanes, zeros_i], pv)

    pltpu.sync_copy(s0_v, cs_hbm.at[0, pl.ds(base, CHUNK)])
    pltpu.sync_copy(s1_v, cs_hbm.at[1, pl.ds(base, CHUNK)])
    hs = []
    for r in range(8):
        hs.append(pltpu.async_copy(tok_v.at[r], rt_hbm.at[si_v.at[r]], sem))
        hs.append(pltpu.async_copy(w8_v.at[pl.ds(r * 64, 64)],
                                   rw8_hbm.at[si_v.at[r]], semw))
    for h in hs:
        h.wait()

    dump = jnp.full((L,), DUMP, jnp.int32)
    widv = jnp.full((L,), 0, jnp.int32) + wid
    my_go = plsc.load_gather(gof_v, [jnp.minimum(widv, E)])
    my_go1 = plsc.load_gather(gof_v, [jnp.minimum(widv + 1, E)])
    my_cnt = plsc.load_gather(cnt_v, [jnp.minimum(widv, E - 1)])
    total_go = plsc.load_gather(gof_v, [jnp.full((L,), E, jnp.int32)])
    for q in range(16):
        idxv = dump
        tail_start = my_go + my_cnt + q * L + lanes
        tail_ok = (widv < E) & (tail_start < my_go1)
        idxv = jnp.where(tail_ok, tail_start, idxv)
        tr_start = total_go + (widv - E) * CHUNK + q * L + lanes
        tr_ok = (widv >= E) & (widv < 2 * E) & (tr_start < P)
        idxv = jnp.where(tr_ok, tr_start, idxv)
        padi_v[q // 4, pl.ds((q % 4) * L, L)] = idxv
        padt_v[q // 4, pl.ds((q % 4) * L, L)] = zeros_i
        plsc.store_scatter(padw_v, [q * L + lanes, zeros_i], zeros_f)
    hs2 = []
    for r in range(4):
        hs2.append(pltpu.async_copy(padt_v.at[r], rt_hbm.at[padi_v.at[r]], sem))
        hs2.append(pltpu.async_copy(padw_v.at[pl.ds(r * 64, 64)],
                                    rw8_hbm.at[padi_v.at[r]], semw))
    for h in hs2:
        h.wait()

    @pl.when(wid == L)
    def _():
        for q in range(NBPAD // L):
            bs = (q * L + lanes) * BM
            acc = jnp.zeros((L,), jnp.int32)
            for e in range(1, E + 1):
                goe = plsc.load_gather(gof_v, [jnp.full((L,), e, jnp.int32)])
                acc = acc + (bs >= goe).astype(jnp.int32)
            be_v[pl.ds(q * L, L)] = jnp.minimum(acc, E - 1)
        pltpu.sync_copy(be_v, be_hbm)


_dispatch = functools.partial(
    pl.kernel,
    _dispatch_body,
    out_type=[
        jax.ShapeDtypeStruct((PX,), jnp.int32),
        jax.ShapeDtypeStruct((PX, 8), jnp.float32),
        jax.ShapeDtypeStruct((2, N), jnp.int32),
        jax.ShapeDtypeStruct((NBPAD,), jnp.int32),
    ],
    mesh=_MESH,
    compiler_params=_SC_PARAMS,
    scratch_types=[
        pltpu.VMEM((NW, E), jnp.int32),
        pltpu.VMEM((CHUNK,), jnp.int32),
        pltpu.VMEM((CHUNK,), jnp.int32),
        pltpu.VMEM((CHUNK,), jnp.float32),
        pltpu.VMEM((CHUNK,), jnp.float32),
        pltpu.VMEM((L,), jnp.int32),
        pltpu.VMEM((L,), jnp.int32),
        pltpu.VMEM((L,), jnp.int32),
        pltpu.VMEM((CHUNK,), jnp.int32),
        pltpu.VMEM((CHUNK,), jnp.int32),
        pltpu.VMEM((8, 64), jnp.int32),
        pltpu.VMEM((8, 64), jnp.int32),
        pltpu.VMEM((512, 8), jnp.float32),
        pltpu.VMEM((4, 64), jnp.int32),
        pltpu.VMEM((4, 64), jnp.int32),
        pltpu.VMEM((256, 8), jnp.float32),
        pltpu.VMEM((NBPAD,), jnp.int32),
        pltpu.SemaphoreType.DMA,
        pltpu.SemaphoreType.DMA,
    ],
)()


GB = 32


def _stream_rows(src_hbm, idx_v, i_base, dst_hbm, d_base, n_rows,
                 buf0, buf1, s0, s1):
    def body(g, carry):
        o = g * (2 * GB)
        h0 = pltpu.async_copy(
            src_hbm.at[idx_v.at[pl.ds(i_base + o, GB)]], buf0, s0)
        h1 = pltpu.async_copy(
            src_hbm.at[idx_v.at[pl.ds(i_base + o + GB, GB)]], buf1, s1)
        h0.wait()
        pltpu.sync_copy(buf0, dst_hbm.at[pl.ds(d_base + o, GB)])
        h1.wait()
        pltpu.sync_copy(buf1, dst_hbm.at[pl.ds(d_base + o + GB, GB)])
        return carry

    lax.fori_loop(0, n_rows // (2 * GB), body, 0)


def _gather_body(xbf_hbm, rt_hbm, xg_hbm, idx_v, buf0, buf1, s0, s1):
    wid = _wid()
    r0 = pl.multiple_of(wid * RPW, CG)
    pltpu.sync_copy(rt_hbm.at[pl.ds(r0, RPW)], idx_v)
    _stream_rows(xbf_hbm, idx_v, 0, xg_hbm, r0, RPW, buf0, buf1, s0, s1)


_gather = functools.partial(
    pl.kernel,
    _gather_body,
    out_type=[jax.ShapeDtypeStruct((P, H), jnp.bfloat16)],
    mesh=_MESH,
    compiler_params=_SC_PARAMS,
    scratch_types=[
        pltpu.VMEM((RPW,), jnp.int32),
        pltpu.VMEM((GB, H), jnp.bfloat16),
        pltpu.VMEM((GB, H), jnp.bfloat16),
        pltpu.SemaphoreType.DMA,
        pltpu.SemaphoreType.DMA,
    ],
)()


def _gmm1_body(be_ref, x_ref, w_ref, b_ref, o_ref, wbf_ref):
    i = pl.program_id(1)
    changed = (i == 0) | (be_ref[i] != be_ref[jnp.maximum(i - 1, 0)])

    @pl.when(changed)
    def _():
        wbf_ref[...] = w_ref[0].astype(jnp.bfloat16)

    acc = lax.dot_general(x_ref[...], wbf_ref[...], (((1,), (0,)), ((), ())),
                          preferred_element_type=jnp.float32)
    acc = acc + b_ref[0]
    o_ref[...] = jnp.maximum(acc, 0.0).astype(jnp.bfloat16)


def _gmm1(be, xg, w1, b1):
    bn = 2048
    gs = pltpu.PrefetchScalarGridSpec(
        num_scalar_prefetch=1,
        grid=(F // bn, NB),
        in_specs=(
            pl.BlockSpec((BM, H), lambda j, i, be: (i, 0)),
            pl.BlockSpec((1, H, bn), lambda j, i, be: (be[i], 0, j)),
            pl.BlockSpec((1, 1, bn), lambda j, i, be: (be[i], 0, j)),
        ),
        out_specs=pl.BlockSpec((BM, bn), lambda j, i, be: (i, j)),
        scratch_shapes=[pltpu.VMEM((H, bn), jnp.bfloat16)],
    )
    return pl.pallas_call(
        _gmm1_body, grid_spec=gs,
        out_shape=jax.ShapeDtypeStruct((P, F), jnp.bfloat16),
    )(be, xg, w1, b1.reshape(E, 1, F))


def _gmm2_body(be_ref, x_ref, w_ref, b_ref, rw_ref, o_ref, wbf_ref):
    i = pl.program_id(1)
    changed = (i == 0) | (be_ref[i] != be_ref[jnp.maximum(i - 1, 0)])

    @pl.when(changed)
    def _():
        wbf_ref[...] = w_ref[0].astype(jnp.bfloat16)

    acc = lax.dot_general(x_ref[...], wbf_ref[...], (((1,), (0,)), ((), ())),
                          preferred_element_type=jnp.float32)
    acc = acc + b_ref[0]
    acc = acc * rw_ref[:, 0:1]
    o_ref[...] = acc.astype(jnp.bfloat16)


def _gmm2(be, h1, w2, b2, rw8):
    bn = 1024
    gs = pltpu.PrefetchScalarGridSpec(
        num_scalar_prefetch=1,
        grid=(H // bn, NB),
        in_specs=(
            pl.BlockSpec((BM, F), lambda j, i, be: (i, 0)),
            pl.BlockSpec((1, F, bn), lambda j, i, be: (be[i], 0, j)),
            pl.BlockSpec((1, 1, bn), lambda j, i, be: (be[i], 0, j)),
            pl.BlockSpec((BM, 8), lambda j, i, be: (i, 0)),
        ),
        out_specs=pl.BlockSpec((BM, bn), lambda j, i, be: (i, j)),
        scratch_shapes=[pltpu.VMEM((F, bn), jnp.bfloat16)],
    )
    return pl.pallas_call(
        _gmm2_body, grid_spec=gs,
        out_shape=jax.ShapeDtypeStruct((P, H), jnp.bfloat16),
    )(be, h1, w2, b2.reshape(E, 1, H), rw8)


def _combine_body(buf_hbm, cs_hbm, a_hbm, b_hbm, i0_v, i1_v,
                  buf0, buf1, s0, s1):
    wid = _wid()
    base = pl.multiple_of(wid * CHUNK, CHUNK)
    pltpu.sync_copy(cs_hbm.at[0, pl.ds(base, CHUNK)], i0_v)
    pltpu.sync_copy(cs_hbm.at[1, pl.ds(base, CHUNK)], i1_v)
    _stream_rows(buf_hbm, i0_v, 0, a_hbm, base, CHUNK, buf0, buf1, s0, s1)
    _stream_rows(buf_hbm, i1_v, 0, b_hbm, base, CHUNK, buf0, buf1, s0, s1)


_combine = functools.partial(
    pl.kernel,
    _combine_body,
    out_type=[
        jax.ShapeDtypeStruct((N, H), jnp.bfloat16),
        jax.ShapeDtypeStruct((N, H), jnp.bfloat16),
    ],
    mesh=_MESH,
    compiler_params=_SC_PARAMS,
    scratch_types=[
        pltpu.VMEM((CHUNK,), jnp.int32),
        pltpu.VMEM((CHUNK,), jnp.int32),
        pltpu.VMEM((GB, H), jnp.bfloat16),
        pltpu.VMEM((GB, H), jnp.bfloat16),
        pltpu.SemaphoreType.DMA,
        pltpu.SemaphoreType.DMA,
    ],
)()


def _add_body(a_ref, b_ref, o_ref):
    o_ref[...] = a_ref[...].astype(jnp.float32) + b_ref[...].astype(jnp.float32)


def _final_add(a, b):
    bmr = 512
    return pl.pallas_call(
        _add_body,
        grid=(N // bmr,),
        in_specs=[
            pl.BlockSpec((bmr, H), lambda i: (i, 0)),
            pl.BlockSpec((bmr, H), lambda i: (i, 0)),
        ],
        out_specs=pl.BlockSpec((bmr, H), lambda i: (i, 0)),
        out_shape=jax.ShapeDtypeStruct((N, H), jnp.float32),
    )(a, b)


def kernel(x, router_w, router_b, w1, b1, w2, b2):
    batch, seq, hidden = x.shape
    flat = x.reshape(N, H)
    lt, xbf = _router(flat, router_w, router_b)
    eidx, probs, hist = _route(lt)
    rt, rw8, cs, be = _dispatch(eidx, probs, hist)
    xg, = _gather(xbf, rt[:P])
    h1 = _gmm1(be, xg, w1, b1)
    buf = _gmm2(be, h1, w2, b2, rw8)
    a, b = _combine(buf, cs)
    out = _final_add(a, b)
    return out.reshape(batch, seq, hidden)
